# baseline (speedup 1.0000x reference)
import functools

import jax
import jax.numpy as jnp
from jax import lax
from jax.experimental import pallas as pl
from jax.experimental.pallas import tpu as pltpu
from jax.experimental.pallas import tpu_sc as plsc

N = 10000
E = 320000
D = 128
DE = 16
H = 5

NC = 2
NS = 16
NW = NC * NS
EPW = E // NW
CHUNK = 400
NCHUNK = EPW // CHUNK
SUB = 80
NSUB = CHUNK // SUB
ZW = 88
AW = 16
ROWS_PT = 632
QUART_PT = ROWS_PT // 4
N_PAD = ROWS_PT * NS


def _tc_prologue(x, wcat, wroot):
    def body(x_ref, wcat_ref, wroot_ref, z_ref, xr_ref):
        xb = x_ref[...]
        z_ref[...] = jnp.dot(xb, wcat_ref[...], preferred_element_type=jnp.float32)
        xr_ref[...] = jnp.dot(xb, wroot_ref[...], preferred_element_type=jnp.float32)

    nb = 10
    rb = N // nb
    return pl.pallas_call(
        body,
        grid=(nb,),
        in_specs=[
            pl.BlockSpec((rb, D), lambda i: (i, 0)),
            pl.BlockSpec((D, ZW), lambda i: (0, 0)),
            pl.BlockSpec((D, AW), lambda i: (0, 0)),
        ],
        out_specs=[
            pl.BlockSpec((rb, ZW), lambda i: (i, 0)),
            pl.BlockSpec((rb, AW), lambda i: (i, 0)),
        ],
        out_shape=[
            jax.ShapeDtypeStruct((N, ZW), jnp.float32),
            jax.ShapeDtypeStruct((N, AW), jnp.float32),
        ],
    )(x, wcat, wroot)


def _sc_edge_kernel(z, srcs, dsts, attr):
    mesh = plsc.VectorSubcoreMesh(core_axis_name="c", subcore_axis_name="s")

    @functools.partial(
        pl.kernel,
        mesh=mesh,
        compiler_params=pltpu.CompilerParams(
            needs_layout_passes=False, use_tc_tiling_on_sc=False),
        out_type=jax.ShapeDtypeStruct((NC, N_PAD, AW), jnp.float32),
        scratch_types=[
            pltpu.VMEM((NCHUNK, NSUB, SUB), jnp.int32),
            pltpu.VMEM((NCHUNK, NSUB, SUB), jnp.int32),
            pltpu.VMEM((CHUNK, DE), jnp.float32),
            pltpu.VMEM((CHUNK, DE), jnp.float32),
            pltpu.VMEM((CHUNK, ZW), jnp.float32),
            pltpu.VMEM((CHUNK, ZW), jnp.float32),
            pltpu.VMEM((CHUNK, AW), jnp.float32),
            pltpu.VMEM((CHUNK, AW), jnp.float32),
            pltpu.VMEM((QUART_PT, AW), jnp.float32),
            pltpu.VMEM_SHARED((N_PAD, AW), jnp.float32),
            pltpu.SemaphoreType.DMA,
            pltpu.SemaphoreType.DMA,
            pltpu.SemaphoreType.DMA,
            pltpu.SemaphoreType.DMA,
            pltpu.SemaphoreType.DMA,
            pltpu.SemaphoreType.DMA,
        ],
    )
    def body(z_hbm, src_hbm, dst_hbm, attr_hbm, out_hbm,
             src_v, dst_v, attr0_v, attr1_v, zg0_v, zg1_v, msg0_v, msg1_v,
             stage_v, acc_sh, sa0, sa1, sg0, sg1, ss0, ss1):
        c = lax.axis_index("c")
        s = lax.axis_index("s")
        wid = c * NS + s
        attr_v = (attr0_v, attr1_v)
        zg_v = (zg0_v, zg1_v)
        msg_v = (msg0_v, msg1_v)
        sa = (sa0, sa1)
        sg = (sg0, sg1)
        ss = (ss0, ss1)

        zero16 = jnp.zeros((AW,), jnp.float32)

        def zstage(i, carry):
            stage_v[i, :] = zero16
            return carry

        lax.fori_loop(0, QUART_PT, zstage, 0)

        def zmsg(i, carry):
            msg0_v[i, :] = zero16
            msg1_v[i, :] = zero16
            return carry

        lax.fori_loop(0, CHUNK, zmsg, 0)

        for q in range(4):
            pltpu.sync_copy(
                stage_v, acc_sh.at[pl.ds(s * ROWS_PT + q * QUART_PT, QUART_PT)])

        pltpu.sync_copy(src_hbm.at[wid], src_v)
        pltpu.sync_copy(dst_hbm.at[wid], dst_v)
        plsc.subcore_barrier()

        iota16 = lax.iota(jnp.int32, 16)

        def fetch(j, b):
            pltpu.async_copy(attr_hbm.at[wid, j], attr_v[b], sa[b])
            for t in range(NSUB):
                pltpu.async_copy(
                    z_hbm.at[src_v.at[j, t]],
                    zg_v[b].at[pl.ds(t * SUB, SUB)], sg[b])

        def chunk_work(j, b, first, last):
            pltpu.make_async_copy(attr_hbm.at[wid, j], attr_v[b], sa[b]).wait()
            for t in range(NSUB):
                pltpu.make_async_copy(
                    z_hbm.at[src_v.at[j, t]],
                    zg_v[b].at[pl.ds(t * SUB, SUB)], sg[b]).wait()

            if not first:
                for t in range(NSUB):
                    pltpu.make_async_copy(
                        msg_v[b].at[pl.ds(t * SUB, SUB)],
                        acc_sh.at[dst_v.at[j, t]], ss[b]).wait()

            def group(g, carry):
                rows = iota16 + g * 16
                accs = [
                    plsc.load_gather(
                        zg_v[b], [rows, jnp.full((16,), 80 + h, jnp.int32)])
                    for h in range(H)
                ]
                for k in range(DE):
                    acol = plsc.load_gather(
                        attr_v[b], [rows, jnp.full((16,), k, jnp.int32)])
                    zcols = [
                        plsc.load_gather(
                            zg_v[b],
                            [rows, jnp.full((16,), h * DE + k, jnp.int32)])
                        for h in range(H)
                    ]
                    accs = [accs[h] + acol * zcols[h] for h in range(H)]
                for h in range(H):
                    plsc.store_scatter(
                        msg_v[b], [rows, jnp.full((16,), h, jnp.int32)],
                        accs[h])
                return carry

            lax.fori_loop(0, CHUNK // 16, group, 0)

            for t in range(NSUB):
                pltpu.async_copy(
                    msg_v[b].at[pl.ds(t * SUB, SUB)],
                    acc_sh.at[dst_v.at[j, t]], ss[b], add=True)

            if not last:
                @pl.when(j + 2 < NCHUNK)
                def _():
                    fetch(j + 2, b)

        fetch(0, 0)
        fetch(1, 1)

        def pair(i, carry):
            @pl.when(i == 0)
            def _():
                chunk_work(0, 0, True, False)
                chunk_work(1, 1, True, False)

            @pl.when(i > 0)
            def _():
                chunk_work(2 * i, 0, False, False)
                chunk_work(2 * i + 1, 1, False, False)
            return carry

        lax.fori_loop(0, NCHUNK // 2, pair, 0)
        chunk_work(NCHUNK - 1, 0, False, True)

        for b in range(2):
            j = NCHUNK - 1 - b
            for t in range(NSUB):
                pltpu.make_async_copy(
                    msg_v[b].at[pl.ds(t * SUB, SUB)],
                    acc_sh.at[dst_v.at[j, t]], ss[b]).wait()

        plsc.subcore_barrier()
        for q in range(4):
            pltpu.sync_copy(
                acc_sh.at[pl.ds(s * ROWS_PT + q * QUART_PT, QUART_PT)], stage_v)
            pltpu.sync_copy(
                stage_v, out_hbm.at[c, pl.ds(s * ROWS_PT + q * QUART_PT, QUART_PT)])

    return body(z, srcs, dsts, attr)


def _tc_epilogue(p0, p1, xr, wl, bl):
    def body(p0_ref, p1_ref, xr_ref, wl_ref, bl_ref, o_ref):
        hh = jnp.maximum(p0_ref[...] + p1_ref[...] + xr_ref[...], 0.0)
        o_ref[...] = jnp.sum(hh * wl_ref[...], axis=1, keepdims=True) + bl_ref[...]

    nb = 10
    rb = N // nb
    return pl.pallas_call(
        body,
        grid=(nb,),
        in_specs=[
            pl.BlockSpec((rb, AW), lambda i: (i, 0)),
            pl.BlockSpec((rb, AW), lambda i: (i, 0)),
            pl.BlockSpec((rb, AW), lambda i: (i, 0)),
            pl.BlockSpec((1, AW), lambda i: (0, 0)),
            pl.BlockSpec((1, 1), lambda i: (0, 0)),
        ],
        out_specs=pl.BlockSpec((rb, 1), lambda i: (i, 0)),
        out_shape=jax.ShapeDtypeStruct((N, 1), jnp.float32),
    )(p0, p1, xr, wl, bl)


def kernel(x, edge_index, edge_attr, W_edge, b_edge, W_root, b_conv, W_lin, b_lin):
    wy = W_edge.reshape(D, H, DE).reshape(D, H * DE)
    bcols = b_edge.reshape(D, H)
    wcat = jnp.concatenate(
        [wy, bcols, jnp.zeros((D, ZW - H * DE - H), jnp.float32)], axis=1)
    wroot = jnp.concatenate(
        [W_root, jnp.zeros((D, AW - H), jnp.float32)], axis=1)

    srcs = edge_index[0].reshape(NW, NCHUNK, NSUB, SUB)
    dsts = edge_index[1].reshape(NW, NCHUNK, NSUB, SUB)
    attr = edge_attr.reshape(NW, NCHUNK, CHUNK, DE)

    z, xr = _tc_prologue(x, wcat, wroot)
    xr = xr + jnp.concatenate([b_conv, jnp.zeros((AW - H,), jnp.float32)])

    p = _sc_edge_kernel(z, srcs, dsts, attr)

    wl = jnp.concatenate([W_lin[0], jnp.zeros((AW - H,), jnp.float32)])
    out = _tc_epilogue(p[0, :N], p[1, :N], xr, wl.reshape(1, AW),
                       b_lin.reshape(1, 1))
    return out


# speedup vs baseline: 1.0616x; 1.0616x over previous
import functools

import jax
import jax.numpy as jnp
from jax import lax
from jax.experimental import pallas as pl
from jax.experimental.pallas import tpu as pltpu
from jax.experimental.pallas import tpu_sc as plsc

N = 10000
E = 320000
D = 128
DE = 16
H = 5

NC = 2
NS = 16
NW = NC * NS
EPW = E // NW
CHUNK = 400
NCHUNK = EPW // CHUNK
SUB = 80
NSUB = CHUNK // SUB
ZW = 88
AW = 16
ROWS_PT = 632
QUART_PT = ROWS_PT // 4
N_PAD = ROWS_PT * NS


def _tc_prologue(x, wcat, wroot):
    def body(x_ref, wcat_ref, wroot_ref, z_ref, xr_ref):
        xb = x_ref[...]
        z_ref[...] = jnp.dot(xb, wcat_ref[...], preferred_element_type=jnp.float32)
        xr_ref[...] = jnp.dot(xb, wroot_ref[...], preferred_element_type=jnp.float32)

    nb = 10
    rb = N // nb
    return pl.pallas_call(
        body,
        grid=(nb,),
        in_specs=[
            pl.BlockSpec((rb, D), lambda i: (i, 0)),
            pl.BlockSpec((D, ZW), lambda i: (0, 0)),
            pl.BlockSpec((D, AW), lambda i: (0, 0)),
        ],
        out_specs=[
            pl.BlockSpec((rb, ZW), lambda i: (i, 0)),
            pl.BlockSpec((rb, AW), lambda i: (i, 0)),
        ],
        out_shape=[
            jax.ShapeDtypeStruct((N, ZW), jnp.float32),
            jax.ShapeDtypeStruct((N, AW), jnp.float32),
        ],
    )(x, wcat, wroot)


def _sc_edge_kernel(z, ei5, attr):
    mesh = plsc.VectorSubcoreMesh(core_axis_name="c", subcore_axis_name="s")

    @functools.partial(
        pl.kernel,
        mesh=mesh,
        compiler_params=pltpu.CompilerParams(
            needs_layout_passes=False, use_tc_tiling_on_sc=False),
        out_type=jax.ShapeDtypeStruct((NC, N_PAD, AW), jnp.float32),
        scratch_types=[
            pltpu.VMEM((NCHUNK, NSUB, SUB), jnp.int32),
            pltpu.VMEM((NCHUNK, NSUB, SUB), jnp.int32),
            pltpu.VMEM((CHUNK, DE), jnp.float32),
            pltpu.VMEM((CHUNK, DE), jnp.float32),
            pltpu.VMEM((CHUNK, ZW), jnp.float32),
            pltpu.VMEM((CHUNK, ZW), jnp.float32),
            pltpu.VMEM((CHUNK, AW), jnp.float32),
            pltpu.VMEM((CHUNK, AW), jnp.float32),
            pltpu.VMEM((QUART_PT, AW), jnp.float32),
            pltpu.VMEM_SHARED((N_PAD, AW), jnp.float32),
            pltpu.SemaphoreType.DMA,
            pltpu.SemaphoreType.DMA,
            pltpu.SemaphoreType.DMA,
            pltpu.SemaphoreType.DMA,
            pltpu.SemaphoreType.DMA,
            pltpu.SemaphoreType.DMA,
        ],
    )
    def body(z_hbm, ei_hbm, attr_hbm, out_hbm,
             src_v, dst_v, attr0_v, attr1_v, zg0_v, zg1_v, msg0_v, msg1_v,
             stage_v, acc_sh, sa0, sa1, sg0, sg1, ss0, ss1):
        c = lax.axis_index("c")
        s = lax.axis_index("s")
        wid = c * NS + s
        attr_v = (attr0_v, attr1_v)
        zg_v = (zg0_v, zg1_v)
        msg_v = (msg0_v, msg1_v)
        sa = (sa0, sa1)
        sg = (sg0, sg1)
        ss = (ss0, ss1)

        zero16 = jnp.zeros((AW,), jnp.float32)

        def zstage(i, carry):
            stage_v[i, :] = zero16
            return carry

        lax.fori_loop(0, QUART_PT, zstage, 0)

        def zmsg(i, carry):
            msg0_v[i, :] = zero16
            msg1_v[i, :] = zero16
            return carry

        lax.fori_loop(0, CHUNK, zmsg, 0)

        for q in range(4):
            pltpu.sync_copy(
                stage_v, acc_sh.at[pl.ds(s * ROWS_PT + q * QUART_PT, QUART_PT)])

        pltpu.sync_copy(ei_hbm.at[0, wid], src_v)
        pltpu.sync_copy(ei_hbm.at[1, wid], dst_v)
        plsc.subcore_barrier()

        iota16 = lax.iota(jnp.int32, 16)

        def fetch(j, b):
            ebase = (wid * NCHUNK + j) * CHUNK
            pltpu.async_copy(attr_hbm.at[pl.ds(ebase, CHUNK)], attr_v[b], sa[b])
            for t in range(NSUB):
                pltpu.async_copy(
                    z_hbm.at[src_v.at[j, t]],
                    zg_v[b].at[pl.ds(t * SUB, SUB)], sg[b])

        def chunk_work(j, b, first, last):
            ebase = (wid * NCHUNK + j) * CHUNK
            pltpu.make_async_copy(
                attr_hbm.at[pl.ds(ebase, CHUNK)], attr_v[b], sa[b]).wait()
            for t in range(NSUB):
                pltpu.make_async_copy(
                    z_hbm.at[src_v.at[j, t]],
                    zg_v[b].at[pl.ds(t * SUB, SUB)], sg[b]).wait()

            if not first:
                for t in range(NSUB):
                    pltpu.make_async_copy(
                        msg_v[b].at[pl.ds(t * SUB, SUB)],
                        acc_sh.at[dst_v.at[j, t]], ss[b]).wait()

            def group(g, carry):
                rows = iota16 + g * 16
                accs = [
                    plsc.load_gather(
                        zg_v[b], [rows, jnp.full((16,), 80 + h, jnp.int32)])
                    for h in range(H)
                ]
                for k in range(DE):
                    acol = plsc.load_gather(
                        attr_v[b], [rows, jnp.full((16,), k, jnp.int32)])
                    zcols = [
                        plsc.load_gather(
                            zg_v[b],
                            [rows, jnp.full((16,), h * DE + k, jnp.int32)])
                        for h in range(H)
                    ]
                    accs = [accs[h] + acol * zcols[h] for h in range(H)]
                for h in range(H):
                    plsc.store_scatter(
                        msg_v[b], [rows, jnp.full((16,), h, jnp.int32)],
                        accs[h])
                return carry

            lax.fori_loop(0, CHUNK // 16, group, 0)

            for t in range(NSUB):
                pltpu.async_copy(
                    msg_v[b].at[pl.ds(t * SUB, SUB)],
                    acc_sh.at[dst_v.at[j, t]], ss[b], add=True)

            if not last:
                @pl.when(j + 2 < NCHUNK)
                def _():
                    fetch(j + 2, b)

        fetch(0, 0)
        fetch(1, 1)

        def pair(i, carry):
            @pl.when(i == 0)
            def _():
                chunk_work(0, 0, True, False)
                chunk_work(1, 1, True, False)

            @pl.when(i > 0)
            def _():
                chunk_work(2 * i, 0, False, False)
                chunk_work(2 * i + 1, 1, False, False)
            return carry

        lax.fori_loop(0, NCHUNK // 2, pair, 0)
        chunk_work(NCHUNK - 1, 0, False, True)

        for b in range(2):
            j = NCHUNK - 1 - b
            for t in range(NSUB):
                pltpu.make_async_copy(
                    msg_v[b].at[pl.ds(t * SUB, SUB)],
                    acc_sh.at[dst_v.at[j, t]], ss[b]).wait()

        plsc.subcore_barrier()
        for q in range(4):
            pltpu.sync_copy(
                acc_sh.at[pl.ds(s * ROWS_PT + q * QUART_PT, QUART_PT)], stage_v)
            pltpu.sync_copy(
                stage_v, out_hbm.at[c, pl.ds(s * ROWS_PT + q * QUART_PT, QUART_PT)])

    return body(z, ei5, attr)


def _tc_epilogue(p, xr, bconv, wl, bl):
    def body(p_ref, xr_ref, bc_ref, wl_ref, bl_ref, o_ref):
        hh = jnp.maximum(
            p_ref[0] + p_ref[1] + xr_ref[...] + bc_ref[...], 0.0)
        o_ref[...] = jnp.sum(hh * wl_ref[...], axis=1, keepdims=True) + bl_ref[...]

    nb = 10
    rb = N // nb
    return pl.pallas_call(
        body,
        grid=(nb,),
        in_specs=[
            pl.BlockSpec((NC, rb, AW), lambda i: (0, i, 0)),
            pl.BlockSpec((rb, AW), lambda i: (i, 0)),
            pl.BlockSpec((1, AW), lambda i: (0, 0)),
            pl.BlockSpec((1, AW), lambda i: (0, 0)),
            pl.BlockSpec((1, 1), lambda i: (0, 0)),
        ],
        out_specs=pl.BlockSpec((rb, 1), lambda i: (i, 0)),
        out_shape=jax.ShapeDtypeStruct((N, 1), jnp.float32),
    )(p, xr, bconv, wl, bl)


def kernel(x, edge_index, edge_attr, W_edge, b_edge, W_root, b_conv, W_lin, b_lin):
    wy = W_edge.reshape(D, H, DE).reshape(D, H * DE)
    bcols = b_edge.reshape(D, H)
    wcat = jnp.concatenate(
        [wy, bcols, jnp.zeros((D, ZW - H * DE - H), jnp.float32)], axis=1)
    wroot = jnp.concatenate(
        [W_root, jnp.zeros((D, AW - H), jnp.float32)], axis=1)

    ei5 = edge_index.reshape(2, NW, NCHUNK, NSUB, SUB)

    z, xr = _tc_prologue(x, wcat, wroot)

    p = _sc_edge_kernel(z, ei5, edge_attr)

    bconv = jnp.concatenate([b_conv, jnp.zeros((AW - H,), jnp.float32)])
    wl = jnp.concatenate([W_lin[0], jnp.zeros((AW - H,), jnp.float32)])
    out = _tc_epilogue(p, xr, bconv.reshape(1, AW), wl.reshape(1, AW),
                       b_lin.reshape(1, 1))
    return out
